# baseline (speedup 1.0000x reference)
import functools

import jax
import jax.numpy as jnp
from jax.experimental import pallas as pl
from jax.experimental.pallas import tpu as pltpu

_P = jax.lax.Precision.DEFAULT


def _dot(a, b):
    return jax.lax.dot_general(
        a, b, (((1,), (0,)), ((), ())),
        preferred_element_type=jnp.float32, precision=_P)


def _spmm(nk, valid_last, gene_ref, adj_ref, x_ref, ws_ref, bs_ref, wf_ref,
          acc_ref, hf_ref):
    k = pl.program_id(0)

    ax = _dot(gene_ref[...], x_ref[...])
    hf_ref[...] = _dot(jnp.tanh(_dot(ax, ws_ref[...]) + bs_ref[...]),
                       wf_ref[...])

    if valid_last < hf_ref.shape[0]:
        @pl.when(k == nk - 1)
        def _mask_ragged():
            hf_ref[valid_last:, :] = jnp.zeros_like(hf_ref[valid_last:, :])
            adj_ref[:, valid_last:] = jnp.zeros_like(adj_ref[:, valid_last:])

    contrib = _dot(adj_ref[...], hf_ref[...])

    @pl.when(k == 0)
    def _init():
        acc_ref[...] = contrib

    @pl.when(k > 0)
    def _accum():
        acc_ref[...] = acc_ref[...] + contrib


def _head(acc_ref, bf_ref, w1_ref, b1_ref, w2_ref, b2_ref, w3_ref, b3_ref,
          out_ref):
    h = jnp.tanh(acc_ref[...] + bf_ref[...])
    h = jnp.tanh(_dot(h, w1_ref[...]) + b1_ref[...])
    h = jnp.tanh(_dot(h, w2_ref[...]) + b2_ref[...])
    out_ref[...] = _dot(h, w3_ref[...]) + b3_ref[...]


def kernel(x, adj, gene_adj, W_s, b_s, W_f, b_f, W1, b1, W2, b2, W3, b3):
    n, f = x.shape
    f1 = W1.shape[1]
    f2 = W2.shape[1]
    nc = W3.shape[1]
    bk = 256
    nk = -(-n // bk)
    valid_last = n - (nk - 1) * bk

    def _const(shape):
        return pl.BlockSpec(shape, lambda i: (0, 0))

    acc = pl.pallas_call(
        functools.partial(_spmm, nk, valid_last),
        grid=(nk,),
        in_specs=[
            pl.BlockSpec((bk, n), lambda k: (k, 0)),
            pl.BlockSpec((n, bk), lambda k: (0, k)),
            _const((n, f)),
            _const((f, f)),
            _const((1, f)),
            _const((f, f)),
        ],
        out_specs=pl.BlockSpec((n, f), lambda k: (0, 0)),
        out_shape=jax.ShapeDtypeStruct((n, f), jnp.float32),
        scratch_shapes=[pltpu.VMEM((bk, f), jnp.float32)],
        compiler_params=pltpu.CompilerParams(
            dimension_semantics=("arbitrary",),
            vmem_limit_bytes=62 * 1024 * 1024,
        ),
    )(gene_adj, adj, x, W_s, b_s.reshape(1, f), W_f)

    bm = 2000 if n % 2000 == 0 else n
    out = pl.pallas_call(
        _head,
        grid=(n // bm,),
        in_specs=[
            pl.BlockSpec((bm, f), lambda i: (i, 0)),
            _const((1, f)),
            _const((f, f1)),
            _const((1, f1)),
            _const((f1, f2)),
            _const((1, f2)),
            _const((f2, nc)),
            _const((1, nc)),
        ],
        out_specs=pl.BlockSpec((bm, nc), lambda i: (i, 0)),
        out_shape=jax.ShapeDtypeStruct((n, nc), jnp.float32),
        compiler_params=pltpu.CompilerParams(
            dimension_semantics=("arbitrary",),
            vmem_limit_bytes=62 * 1024 * 1024,
        ),
    )(acc, b_f.reshape(1, f), W1, b1.reshape(1, f1),
      W2, b2.reshape(1, f2), W3, b3.reshape(1, nc))
    return out


# speedup vs baseline: 1.0138x; 1.0138x over previous
import functools

import jax
import jax.numpy as jnp
from jax.experimental import pallas as pl
from jax.experimental.pallas import tpu as pltpu

_P = jax.lax.Precision.DEFAULT
_BM = 200


def _dot(a, b):
    return jax.lax.dot_general(
        a, b, (((1,), (0,)), ((), ())),
        preferred_element_type=jnp.float32, precision=_P)


def _hf_block(g, x, ws, bs, wf):
    h = jnp.tanh(_dot(_dot(g, x), ws) + bs)
    return _dot(h, wf)


def _mlp_block(a, hf, bf, w1, b1, w2, b2, w3, b3):
    h = jnp.tanh(_dot(a, hf) + bf)
    h = jnp.tanh(_dot(h, w1) + b1)
    h = jnp.tanh(_dot(h, w2) + b2)
    return _dot(h, w3) + b3


def _stage1_dual(ga_ref, gb_ref, x_ref, ws_ref, bs_ref, wf_ref,
                 hfa_ref, hfb_ref):
    hfa_ref[...] = _hf_block(ga_ref[...], x_ref[...], ws_ref[...],
                             bs_ref[...], wf_ref[...])
    hfb_ref[...] = _hf_block(gb_ref[...], x_ref[...], ws_ref[...],
                             bs_ref[...], wf_ref[...])


def _stage2_dual(half, aa_ref, ab_ref, hfa_ref, hfb_ref, bf_ref, w1_ref,
                 b1_ref, w2_ref, b2_ref, w3_ref, b3_ref,
                 oa_ref, ob_ref, hf_ref):
    i = pl.program_id(0)

    @pl.when(i == 0)
    def _assemble():
        hf_ref[:half, :] = hfa_ref[...]
        hf_ref[half:, :] = hfb_ref[...]

    hf = hf_ref[...]
    oa_ref[...] = _mlp_block(aa_ref[...], hf, bf_ref[...], w1_ref[...],
                             b1_ref[...], w2_ref[...], b2_ref[...],
                             w3_ref[...], b3_ref[...])
    ob_ref[...] = _mlp_block(ab_ref[...], hf, bf_ref[...], w1_ref[...],
                             b1_ref[...], w2_ref[...], b2_ref[...],
                             w3_ref[...], b3_ref[...])


def _stage1_single(g_ref, x_ref, ws_ref, bs_ref, wf_ref, hf_ref):
    hf_ref[...] = _hf_block(g_ref[...], x_ref[...], ws_ref[...],
                            bs_ref[...], wf_ref[...])


def _stage2_single(a_ref, hf_ref, bf_ref, w1_ref, b1_ref, w2_ref, b2_ref,
                   w3_ref, b3_ref, out_ref):
    out_ref[...] = _mlp_block(a_ref[...], hf_ref[...], bf_ref[...],
                              w1_ref[...], b1_ref[...], w2_ref[...],
                              b2_ref[...], w3_ref[...], b3_ref[...])


def _pick_bm(n):
    for bm in (400, 256, 200, 128, 100, 80, 40, 8):
        if n % bm == 0:
            return bm
    return n


def kernel(x, adj, gene_adj, W_s, b_s, W_f, b_f, W1, b1, W2, b2, W3, b3):
    n, f = x.shape
    f1 = W1.shape[1]
    f2 = W2.shape[1]
    nc = W3.shape[1]
    cparams = pltpu.CompilerParams(
        dimension_semantics=("arbitrary",),
        vmem_limit_bytes=62 * 1024 * 1024,
    )

    def _const(shape):
        return pl.BlockSpec(shape, lambda i: (0, 0))

    if n % (2 * _BM) != 0:
        bm = _pick_bm(n)
        grid = (n // bm,)
        hf = pl.pallas_call(
            _stage1_single,
            grid=grid,
            in_specs=[
                pl.BlockSpec((bm, n), lambda i: (i, 0)),
                _const((n, f)), _const((f, f)), _const((1, f)),
                _const((f, f)),
            ],
            out_specs=pl.BlockSpec((bm, f), lambda i: (i, 0)),
            out_shape=jax.ShapeDtypeStruct((n, f), jnp.float32),
            compiler_params=cparams,
        )(gene_adj, x, W_s, b_s.reshape(1, f), W_f)
        return pl.pallas_call(
            _stage2_single,
            grid=grid,
            in_specs=[
                pl.BlockSpec((bm, n), lambda i: (i, 0)),
                _const((n, f)), _const((1, f)), _const((f, f1)),
                _const((1, f1)), _const((f1, f2)), _const((1, f2)),
                _const((f2, nc)), _const((1, nc)),
            ],
            out_specs=pl.BlockSpec((bm, nc), lambda i: (i, 0)),
            out_shape=jax.ShapeDtypeStruct((n, nc), jnp.float32),
            compiler_params=cparams,
        )(adj, hf, b_f.reshape(1, f), W1, b1.reshape(1, f1),
          W2, b2.reshape(1, f2), W3, b3.reshape(1, nc))

    half = n // 2
    g = half // _BM

    hfa, hfb = pl.pallas_call(
        _stage1_dual,
        grid=(g,),
        in_specs=[
            pl.BlockSpec((_BM, n), lambda i: (i, 0)),
            pl.BlockSpec((_BM, n), lambda i, g=g: (g + i, 0)),
            _const((n, f)), _const((f, f)), _const((1, f)), _const((f, f)),
        ],
        out_specs=[
            pl.BlockSpec((_BM, f), lambda i: (i, 0)),
            pl.BlockSpec((_BM, f), lambda i: (i, 0)),
        ],
        out_shape=[
            jax.ShapeDtypeStruct((half, f), jnp.float32),
            jax.ShapeDtypeStruct((half, f), jnp.float32),
        ],
        compiler_params=cparams,
    )(gene_adj, gene_adj, x, W_s, b_s.reshape(1, f), W_f)

    outa, outb = pl.pallas_call(
        functools.partial(_stage2_dual, half),
        grid=(g,),
        in_specs=[
            pl.BlockSpec((_BM, n), lambda i: (i, 0)),
            pl.BlockSpec((_BM, n), lambda i, g=g: (g + i, 0)),
            _const((half, f)), _const((half, f)),
            _const((1, f)), _const((f, f1)), _const((1, f1)),
            _const((f1, f2)), _const((1, f2)), _const((f2, nc)),
            _const((1, nc)),
        ],
        out_specs=[
            pl.BlockSpec((_BM, nc), lambda i: (i, 0)),
            pl.BlockSpec((_BM, nc), lambda i: (i, 0)),
        ],
        out_shape=[
            jax.ShapeDtypeStruct((half, nc), jnp.float32),
            jax.ShapeDtypeStruct((half, nc), jnp.float32),
        ],
        scratch_shapes=[pltpu.VMEM((n, f), jnp.float32)],
        compiler_params=cparams,
    )(adj, adj, hfa, hfb, b_f.reshape(1, f), W1, b1.reshape(1, f1),
      W2, b2.reshape(1, f2), W3, b3.reshape(1, nc))
    return jnp.concatenate([outa, outb], axis=0)


# speedup vs baseline: 1.1352x; 1.1198x over previous
import jax
import jax.numpy as jnp
from jax.experimental import pallas as pl
from jax.experimental.pallas import tpu as pltpu

_P = jax.lax.Precision.DEFAULT


def _dot(a, b):
    return jax.lax.dot_general(
        a, b, (((1,), (0,)), ((), ())),
        preferred_element_type=jnp.float32, precision=_P)


def _stage1(gene_ref, x_ref, ws_ref, bs_ref, wf_ref, hf_ref):
    ax = _dot(gene_ref[...], x_ref[...])
    h = jnp.tanh(_dot(ax, ws_ref[...]) + bs_ref[...])
    hf_ref[...] = _dot(h, wf_ref[...])


def _stage2(adj_ref, hf_ref, bf_ref, w1_ref, b1_ref, w2_ref, b2_ref,
            w3_ref, b3_ref, out_ref):
    acc = _dot(adj_ref[...], hf_ref[...])
    h = jnp.tanh(acc + bf_ref[...])
    h = jnp.tanh(_dot(h, w1_ref[...]) + b1_ref[...])
    h = jnp.tanh(_dot(h, w2_ref[...]) + b2_ref[...])
    out_ref[...] = _dot(h, w3_ref[...]) + b3_ref[...]


def _pick_bm(n):
    for bm in (400, 256, 200, 128, 100, 80, 40, 8):
        if n % bm == 0:
            return bm
    return n


def kernel(x, adj, gene_adj, W_s, b_s, W_f, b_f, W1, b1, W2, b2, W3, b3):
    n, f = x.shape
    f1 = W1.shape[1]
    f2 = W2.shape[1]
    nc = W3.shape[1]
    bm = _pick_bm(n)
    grid = (n // bm,)
    cparams = pltpu.CompilerParams(
        dimension_semantics=("parallel",),
        vmem_limit_bytes=110 * 1024 * 1024,
    )

    def _const(shape):
        return pl.BlockSpec(shape, lambda i: (0, 0))

    hf = pl.pallas_call(
        _stage1,
        grid=grid,
        in_specs=[
            pl.BlockSpec((bm, n), lambda i: (i, 0)),
            _const((n, f)),
            _const((f, f)),
            _const((1, f)),
            _const((f, f)),
        ],
        out_specs=pl.BlockSpec((bm, f), lambda i: (i, 0)),
        out_shape=jax.ShapeDtypeStruct((n, f), jnp.float32),
        compiler_params=cparams,
    )(gene_adj, x, W_s, b_s.reshape(1, f), W_f)

    out = pl.pallas_call(
        _stage2,
        grid=grid,
        in_specs=[
            pl.BlockSpec((bm, n), lambda i: (i, 0)),
            _const((n, f)),
            _const((1, f)),
            _const((f, f1)),
            _const((1, f1)),
            _const((f1, f2)),
            _const((1, f2)),
            _const((f2, nc)),
            _const((1, nc)),
        ],
        out_specs=pl.BlockSpec((bm, nc), lambda i: (i, 0)),
        out_shape=jax.ShapeDtypeStruct((n, nc), jnp.float32),
        compiler_params=cparams,
    )(adj, hf, b_f.reshape(1, f), W1, b1.reshape(1, f1),
      W2, b2.reshape(1, f2), W3, b3.reshape(1, nc))
    return out
